# baseline (speedup 1.0000x reference)
import functools

import jax
import jax.numpy as jnp
from jax import lax
from jax.experimental import pallas as pl
from jax.experimental.pallas import tpu as pltpu
from jax.experimental.pallas import tpu_sc as plsc

D = 128
CH = 128
NC = 2
NS = 16
NW = NC * NS
ACC_ROWS = 10112


def _sc_aggregate(table, eb, zeros_blk):
    nch = eb.shape[0]
    base_trips, rem = divmod(nch, NW)
    zrows = ACC_ROWS // NS

    mesh = plsc.VectorSubcoreMesh(
        core_axis_name="c", subcore_axis_name="s",
        num_cores=NC, num_subcores=NS)

    @functools.partial(
        pl.kernel,
        out_type=jax.ShapeDtypeStruct((NC, ACC_ROWS, D), jnp.float32),
        mesh=mesh,
        scratch_types=[
            pltpu.VMEM_SHARED((ACC_ROWS, D), jnp.float32),
            pltpu.VMEM((3, CH, D), jnp.float32),
            pltpu.VMEM((4, 2, CH), jnp.int32),
            pltpu.SemaphoreType.DMA,
            pltpu.SemaphoreType.DMA,
            pltpu.SemaphoreType.DMA,
            pltpu.SemaphoreType.DMA,
            pltpu.SemaphoreType.DMA,
            pltpu.SemaphoreType.DMA,
            pltpu.SemaphoreType.DMA,
            pltpu.SemaphoreType.DMA,
        ],
    )
    def agg(table_hbm, eb_hbm, zeros_hbm, out_hbm,
            acc, rows, ib, isem, g0, g1, g2, s0, s1, s2, wsem):
        gsem = (g0, g1, g2)
        ssem = (s0, s1, s2)
        cid = lax.axis_index("c")
        sid = lax.axis_index("s")
        wid = sid * NC + cid

        zchunks = [(k * CH, CH) for k in range(zrows // CH)]
        tail = zrows - (zrows // CH) * CH
        if tail:
            zchunks.append(((zrows // CH) * CH, tail))

        ntr = base_trips + jnp.where(wid < rem, 1, 0).astype(jnp.int32)

        def i_copy(i):
            return pltpu.make_async_copy(
                eb_hbm.at[wid + i * NW], ib.at[lax.rem(i, 4)], isem)

        def g_copy(i, s):
            return pltpu.make_async_copy(
                table_hbm.at[ib.at[lax.rem(i, 4), 0]],
                rows.at[lax.rem(i, 3)], gsem[s])

        def s_start(i, s):
            pltpu.async_copy(
                rows.at[lax.rem(i, 3)], acc.at[ib.at[lax.rem(i, 4), 1]],
                ssem[s], add=True)

        def s_wait(i, s):
            pltpu.make_async_copy(
                rows.at[lax.rem(i, 3)], acc.at[ib.at[lax.rem(i, 4), 1]],
                ssem[s]).wait()

        for (r0, sz) in zchunks:
            pltpu.async_copy(zeros_hbm.at[pl.ds(0, sz)],
                             acc.at[pl.ds(sid * zrows + r0, sz)], wsem)
        for k in range(2):
            @pl.when(ntr > k)
            def _(k=k):
                i_copy(k).start()
                i_copy(k).wait()
                g_copy(k, k).start()

        @pl.when(ntr > 2)
        def _():
            i_copy(2).start()
        for (r0, sz) in zchunks:
            pltpu.make_async_copy(
                zeros_hbm.at[pl.ds(0, sz)],
                acc.at[pl.ds(sid * zrows + r0, sz)], wsem).wait()
        plsc.subcore_barrier()

        def chunk_step(i, u):
            g_copy(i, u).wait()
            s_start(i, u)
            j = i + 2

            @pl.when(j < ntr)
            def _():
                i_copy(j).wait()

            if u == 0:
                @pl.when(i >= 1)
                def _():
                    s_wait(i - 1, (u + 2) % 3)
            else:
                s_wait(i - 1, (u + 2) % 3)

            @pl.when(i + 3 < ntr)
            def _():
                i_copy(i + 3).start()

            @pl.when(j < ntr)
            def _():
                g_copy(j, (u + 2) % 3).start()

        def body(t, carry):
            for u in range(3):
                i = t * 3 + u

                @pl.when(i < ntr)
                def _(i=i, u=u):
                    chunk_step(i, u)
            return carry

        lax.fori_loop(0, (ntr + 2) // 3, body, 0)
        for u in range(3):
            @pl.when(lax.rem(ntr - 1, 3) == u)
            def _(u=u):
                s_wait(ntr - 1, u)
        plsc.subcore_barrier()

        for (r0, sz) in zchunks:
            ra = sid * zrows + r0
            pltpu.async_copy(acc.at[pl.ds(ra, sz)],
                             out_hbm.at[cid, pl.ds(ra, sz)], wsem)
        for (r0, sz) in zchunks:
            ra = sid * zrows + r0
            pltpu.make_async_copy(
                acc.at[pl.ds(ra, sz)],
                out_hbm.at[cid, pl.ds(ra, sz)], wsem).wait()

    return agg(table, eb, zeros_blk)


def _tc_layer(agg2, hprev, wt, wlt, bias, apply_elu, n):
    bn = 2000
    dout = wt.shape[1]

    def body(p0_r, p1_r, hp_r, wt_r, wlt_r, b_r, o_r):
        aggm = p0_r[0] + p1_r[0]
        y = jnp.dot(aggm, wt_r[...], preferred_element_type=jnp.float32)
        y = y + jnp.dot(hp_r[...], wlt_r[...], preferred_element_type=jnp.float32)
        y = y + b_r[...]
        if apply_elu:
            y = jnp.where(y > 0, y, jnp.exp(jnp.minimum(y, 0.0)) - 1.0)
        o_r[...] = y

    return pl.pallas_call(
        body,
        grid=(n // bn,),
        in_specs=[
            pl.BlockSpec((1, bn, D), lambda i: (0, i, 0)),
            pl.BlockSpec((1, bn, D), lambda i: (1, i, 0)),
            pl.BlockSpec((bn, D), lambda i: (i, 0)),
            pl.BlockSpec((D, dout), lambda i: (0, 0)),
            pl.BlockSpec((D, dout), lambda i: (0, 0)),
            pl.BlockSpec((1, dout), lambda i: (0, 0)),
        ],
        out_specs=pl.BlockSpec((bn, dout), lambda i: (i, 0)),
        out_shape=jax.ShapeDtypeStruct((n, dout), jnp.float32),
    )(agg2, agg2, hprev, wt, wlt, bias)


def kernel(x, edge_index, W1, b1, W2, b2, W3, b3,
           Wl1, bl1, Wl2, bl2, Wl3, bl3):
    n = x.shape[0]
    e = edge_index.shape[1]
    eb = edge_index.reshape(2, e // CH, CH).transpose(1, 0, 2)
    zeros_blk = jnp.zeros((CH, D), jnp.float32)

    agg1 = _sc_aggregate(x, eb, zeros_blk)
    h1 = _tc_layer(agg1, x, W1.T, Wl1.T, (b1 + bl1)[None, :], True, n)
    agg2 = _sc_aggregate(h1, eb, zeros_blk)
    h2 = _tc_layer(agg2, h1, W2.T, Wl2.T, (b2 + bl2)[None, :], True, n)
    agg3 = _sc_aggregate(h2, eb, zeros_blk)
    return _tc_layer(agg3, h2, W3.T, Wl3.T, (b3 + bl3)[None, :], False, n)


# speedup vs baseline: 1.0011x; 1.0011x over previous
import functools

import jax
import jax.numpy as jnp
from jax import lax
from jax.experimental import pallas as pl
from jax.experimental.pallas import tpu as pltpu
from jax.experimental.pallas import tpu_sc as plsc

D = 128
CH = 128
NC = 2
NS = 16
NW = NC * NS
ACC_ROWS = 10112


def _sc_aggregate(table, eb, zeros_blk):
    nch = eb.shape[0]
    base_trips, rem = divmod(nch, NW)
    zrows = ACC_ROWS // NS

    mesh = plsc.VectorSubcoreMesh(
        core_axis_name="c", subcore_axis_name="s",
        num_cores=NC, num_subcores=NS)

    @functools.partial(
        pl.kernel,
        out_type=jax.ShapeDtypeStruct((NC, ACC_ROWS, D), jnp.float32),
        mesh=mesh,
        scratch_types=[
            pltpu.VMEM_SHARED((ACC_ROWS, D), jnp.float32),
            pltpu.VMEM((3, CH, D), jnp.float32),
            pltpu.VMEM((4, 2, CH), jnp.int32),
            pltpu.SemaphoreType.DMA,
            pltpu.SemaphoreType.DMA,
            pltpu.SemaphoreType.DMA,
            pltpu.SemaphoreType.DMA,
            pltpu.SemaphoreType.DMA,
            pltpu.SemaphoreType.DMA,
            pltpu.SemaphoreType.DMA,
            pltpu.SemaphoreType.DMA,
        ],
    )
    def agg(table_hbm, eb_hbm, zeros_hbm, out_hbm,
            acc, rows, ib, isem, g0, g1, g2, s0, s1, s2, wsem):
        gsem = (g0, g1, g2)
        ssem = (s0, s1, s2)
        cid = lax.axis_index("c")
        sid = lax.axis_index("s")
        wid = sid * NC + cid

        zchunks = [(k * CH, CH) for k in range(zrows // CH)]
        tail = zrows - (zrows // CH) * CH
        if tail:
            zchunks.append(((zrows // CH) * CH, tail))

        ntr = base_trips + jnp.where(wid < rem, 1, 0).astype(jnp.int32)

        def i_copy(i):
            return pltpu.make_async_copy(
                eb_hbm.at[wid + i * NW], ib.at[lax.rem(i, 4)], isem)

        def g_copy(i, s):
            return pltpu.make_async_copy(
                table_hbm.at[ib.at[lax.rem(i, 4), 0]],
                rows.at[lax.rem(i, 3)], gsem[s])

        def s_start(i, s):
            pltpu.async_copy(
                rows.at[lax.rem(i, 3)], acc.at[ib.at[lax.rem(i, 4), 1]],
                ssem[s], add=True)

        def s_wait(i, s):
            pltpu.make_async_copy(
                rows.at[lax.rem(i, 3)], acc.at[ib.at[lax.rem(i, 4), 1]],
                ssem[s]).wait()

        for (r0, sz) in zchunks:
            pltpu.async_copy(zeros_hbm.at[pl.ds(0, sz)],
                             acc.at[pl.ds(sid * zrows + r0, sz)], wsem)
        for k in range(2):
            @pl.when(ntr > k)
            def _(k=k):
                i_copy(k).start()
                i_copy(k).wait()
                g_copy(k, k).start()

        @pl.when(ntr > 2)
        def _():
            i_copy(2).start()
        for (r0, sz) in zchunks:
            pltpu.make_async_copy(
                zeros_hbm.at[pl.ds(0, sz)],
                acc.at[pl.ds(sid * zrows + r0, sz)], wsem).wait()
        plsc.subcore_barrier()

        def chunk_step(i, u):
            g_copy(i, u).wait()
            s_start(i, u)
            j = i + 2

            @pl.when(j < ntr)
            def _():
                i_copy(j).wait()

            if u == 0:
                @pl.when(i >= 1)
                def _():
                    s_wait(i - 1, (u + 2) % 3)
            else:
                s_wait(i - 1, (u + 2) % 3)

            @pl.when(j < ntr)
            def _():
                g_copy(j, (u + 2) % 3).start()

            @pl.when(i + 3 < ntr)
            def _():
                i_copy(i + 3).start()

        def body(t, carry):
            for u in range(3):
                i = t * 3 + u

                @pl.when(i < ntr)
                def _(i=i, u=u):
                    chunk_step(i, u)
            return carry

        lax.fori_loop(0, (ntr + 2) // 3, body, 0)
        for u in range(3):
            @pl.when(lax.rem(ntr - 1, 3) == u)
            def _(u=u):
                s_wait(ntr - 1, u)
        plsc.subcore_barrier()

        for (r0, sz) in zchunks:
            ra = sid * zrows + r0
            pltpu.async_copy(acc.at[pl.ds(ra, sz)],
                             out_hbm.at[cid, pl.ds(ra, sz)], wsem)
        for (r0, sz) in zchunks:
            ra = sid * zrows + r0
            pltpu.make_async_copy(
                acc.at[pl.ds(ra, sz)],
                out_hbm.at[cid, pl.ds(ra, sz)], wsem).wait()

    return agg(table, eb, zeros_blk)


def _tc_layer(agg2, hprev, wt, wlt, bias, apply_elu, n):
    bn = 2000
    dout = wt.shape[1]

    def body(p_r, hp_r, wt_r, wlt_r, b_r, o_r):
        aggm = p_r[0] + p_r[1]
        y = jnp.dot(aggm, wt_r[...], preferred_element_type=jnp.float32)
        y = y + jnp.dot(hp_r[...], wlt_r[...], preferred_element_type=jnp.float32)
        y = y + b_r[...]
        if apply_elu:
            y = jnp.where(y > 0, y, jnp.exp(jnp.minimum(y, 0.0)) - 1.0)
        o_r[...] = y

    return pl.pallas_call(
        body,
        grid=(n // bn,),
        in_specs=[
            pl.BlockSpec((NC, bn, D), lambda i: (0, i, 0)),
            pl.BlockSpec((bn, D), lambda i: (i, 0)),
            pl.BlockSpec((D, dout), lambda i: (0, 0)),
            pl.BlockSpec((D, dout), lambda i: (0, 0)),
            pl.BlockSpec((1, dout), lambda i: (0, 0)),
        ],
        out_specs=pl.BlockSpec((bn, dout), lambda i: (i, 0)),
        out_shape=jax.ShapeDtypeStruct((n, dout), jnp.float32),
    )(agg2, hprev, wt, wlt, bias)


def kernel(x, edge_index, W1, b1, W2, b2, W3, b3,
           Wl1, bl1, Wl2, bl2, Wl3, bl3):
    n = x.shape[0]
    e = edge_index.shape[1]
    eb = edge_index.reshape(2, e // CH, CH).transpose(1, 0, 2)
    zeros_blk = jnp.zeros((CH, D), jnp.float32)

    agg1 = _sc_aggregate(x, eb, zeros_blk)
    h1 = _tc_layer(agg1, x, W1.T, Wl1.T, (b1 + bl1)[None, :], True, n)
    agg2 = _sc_aggregate(h1, eb, zeros_blk)
    h2 = _tc_layer(agg2, h1, W2.T, Wl2.T, (b2 + bl2)[None, :], True, n)
    agg3 = _sc_aggregate(h2, eb, zeros_blk)
    return _tc_layer(agg3, h2, W3.T, Wl3.T, (b3 + bl3)[None, :], False, n)
